# baseline (speedup 1.0000x reference)
import functools

import jax
import jax.numpy as jnp
from jax.experimental import pallas as pl
from jax.experimental.pallas import tpu as pltpu

T, H, E, D = 64, 2048, 64, 768


def _moe_expert_kernel(hs_ref, rw_ref, gu_ref, dn_ref, out_ref):
    e = pl.program_id(0)
    gu = jnp.dot(
        hs_ref[...],
        gu_ref[0].astype(jnp.bfloat16),
        preferred_element_type=jnp.float32,
    )
    gate = gu[:, :D]
    up = gu[:, D:]
    gated = up * (gate * jax.nn.sigmoid(gate))
    o = jnp.dot(
        gated.astype(jnp.bfloat16),
        dn_ref[0].astype(jnp.bfloat16),
        preferred_element_type=jnp.float32,
    )
    w = rw_ref[e, 0, :]
    contrib = o * w[:, None]

    @pl.when(e == 0)
    def _init():
        out_ref[...] = contrib

    @pl.when(e != 0)
    def _accum():
        out_ref[...] += contrib


@functools.partial(jax.jit, static_argnames=("interpret",))
def _moe(hidden_states, routing_weights, gate_up_proj, down_proj, interpret=False):
    rw_t = routing_weights.T.reshape(E, 1, T)
    hs16 = hidden_states.astype(jnp.bfloat16)
    return pl.pallas_call(
        _moe_expert_kernel,
        grid=(E,),
        in_specs=[
            pl.BlockSpec((T, H), lambda e: (0, 0)),
            pl.BlockSpec((E, 1, T), lambda e: (0, 0, 0)),
            pl.BlockSpec((1, H, 2 * D), lambda e: (e, 0, 0)),
            pl.BlockSpec((1, D, H), lambda e: (e, 0, 0)),
        ],
        out_specs=pl.BlockSpec((T, H), lambda e: (0, 0)),
        out_shape=jax.ShapeDtypeStruct((T, H), jnp.float32),
        compiler_params=pltpu.CompilerParams(
            dimension_semantics=("arbitrary",),
        ),
        interpret=interpret,
    )(hs16, rw_t, gate_up_proj, down_proj)


def kernel(hidden_states, routing_weights, router_indices, gate_up_proj, down_proj):
    del router_indices
    out = _moe(hidden_states, routing_weights, gate_up_proj, down_proj)
    return out.reshape(T, 1, H)


# speedup vs baseline: 1.0072x; 1.0072x over previous
import functools

import jax
import jax.numpy as jnp
from jax.experimental import pallas as pl
from jax.experimental.pallas import tpu as pltpu

T, H, E, D = 64, 2048, 64, 768


def _moe_expert_kernel(hs_ref, rw_ref, gu_ref, dn_ref, out_ref):
    e = pl.program_id(0)
    gu = jnp.dot(
        hs_ref[...],
        gu_ref[0].astype(jnp.bfloat16),
        preferred_element_type=jnp.float32,
    )
    gate = gu[:, :D]
    up = gu[:, D:]
    gated = up * (gate * jax.nn.sigmoid(gate))
    o = jnp.dot(
        gated.astype(jnp.bfloat16),
        dn_ref[0].astype(jnp.bfloat16),
        preferred_element_type=jnp.float32,
    )
    w = rw_ref[0, 0, :]
    contrib = o * w[:, None]

    @pl.when(e == 0)
    def _init():
        out_ref[...] = contrib

    @pl.when(e != 0)
    def _accum():
        out_ref[...] += contrib


@functools.partial(jax.jit, static_argnames=("interpret",))
def _moe(hidden_states, routing_weights, gate_up_proj, down_proj, interpret=False):
    rw_t = routing_weights.T.reshape(E, 1, T)
    hs16 = hidden_states.astype(jnp.bfloat16)
    return pl.pallas_call(
        _moe_expert_kernel,
        grid=(E,),
        in_specs=[
            pl.BlockSpec((T, H), lambda e: (0, 0)),
            pl.BlockSpec((1, 1, T), lambda e: (e, 0, 0)),
            pl.BlockSpec((1, H, 2 * D), lambda e: (e, 0, 0)),
            pl.BlockSpec((1, D, H), lambda e: (e, 0, 0)),
        ],
        out_specs=pl.BlockSpec((T, H), lambda e: (0, 0)),
        out_shape=jax.ShapeDtypeStruct((T, H), jnp.float32),
        compiler_params=pltpu.CompilerParams(
            dimension_semantics=("arbitrary",),
        ),
        interpret=interpret,
    )(hs16, rw_t, gate_up_proj, down_proj)


def kernel(hidden_states, routing_weights, router_indices, gate_up_proj, down_proj):
    del router_indices
    out = _moe(hidden_states, routing_weights, gate_up_proj, down_proj)
    return out.reshape(T, 1, H)


# speedup vs baseline: 1.0144x; 1.0072x over previous
import functools

import jax
import jax.numpy as jnp
from jax.experimental import pallas as pl
from jax.experimental.pallas import tpu as pltpu

T, H, E, D = 64, 2048, 64, 768


def _moe_expert_kernel(hs_ref, rw_ref, gu_ref, dn_ref, out_ref):
    e = pl.program_id(0)
    gu = jnp.dot(
        hs_ref[...].astype(jnp.bfloat16),
        gu_ref[0].astype(jnp.bfloat16),
        preferred_element_type=jnp.float32,
    )
    gate = gu[:, :D]
    up = gu[:, D:]
    w = rw_ref[0, 0, :]
    gated = (up * w[:, None]) * (gate * jax.nn.sigmoid(gate))
    contrib = jnp.dot(
        gated.astype(jnp.bfloat16),
        dn_ref[0].astype(jnp.bfloat16),
        preferred_element_type=jnp.float32,
    )

    @pl.when(e == 0)
    def _init():
        out_ref[...] = contrib

    @pl.when(e != 0)
    def _accum():
        out_ref[...] += contrib


@functools.partial(jax.jit, static_argnames=("interpret",))
def _moe(hidden_states, routing_weights, gate_up_proj, down_proj, interpret=False):
    rw_t = routing_weights.T.reshape(E, 1, T)
    return pl.pallas_call(
        _moe_expert_kernel,
        grid=(E,),
        in_specs=[
            pl.BlockSpec((T, H), lambda e: (0, 0)),
            pl.BlockSpec((1, 1, T), lambda e: (e, 0, 0)),
            pl.BlockSpec((1, H, 2 * D), lambda e: (e, 0, 0)),
            pl.BlockSpec((1, D, H), lambda e: (e, 0, 0)),
        ],
        out_specs=pl.BlockSpec((T, H), lambda e: (0, 0)),
        out_shape=jax.ShapeDtypeStruct((T, H), jnp.float32),
        compiler_params=pltpu.CompilerParams(
            dimension_semantics=("arbitrary",),
        ),
        interpret=interpret,
    )(hidden_states, rw_t, gate_up_proj, down_proj)


def kernel(hidden_states, routing_weights, router_indices, gate_up_proj, down_proj):
    del router_indices
    out = _moe(hidden_states, routing_weights, gate_up_proj, down_proj)
    return out.reshape(T, 1, H)
